# baseline (speedup 1.0000x reference)
import jax
import jax.numpy as jnp
from jax.experimental import pallas as pl
from jax.experimental.pallas import tpu as pltpu

_TN = 16


def _moe_kernel(x_ref, w_ref, b_ref, we_ref, wn_ref, u_ref, o_ref):
    C, TN, P = x_ref.shape[1:]
    TM = TN * P
    xb = x_ref[0].astype(jnp.bfloat16).reshape(C, TM)
    y = jax.lax.dot(
        w_ref[...], xb, preferred_element_type=jnp.float32,
    )

    el = jax.lax.dot(we_ref[...], xb, preferred_element_type=jnp.float32)
    nl = jax.lax.dot(wn_ref[...], xb, preferred_element_type=jnp.float32)
    nexp = jnp.exp(nl - jnp.max(nl, axis=0, keepdims=True))
    noise = u_ref[0].reshape(-1, TM) * (nexp / jnp.sum(nexp, axis=0, keepdims=True))
    logits = el + noise
    drop = jnp.argmin(logits, axis=0)[None, :]
    keep = jax.lax.broadcasted_iota(jnp.int32, logits.shape, 0) != drop
    mexp = jnp.where(keep, jnp.exp(logits - jnp.max(logits, axis=0, keepdims=True)), 0.0)
    gates = mexp / jnp.sum(mexp, axis=0, keepdims=True)
    gsum = jnp.sum(gates, axis=0, keepdims=True)

    o_ref[0] = (gsum * y + b_ref[...]).reshape(o_ref.shape[1], TN, P)


def kernel(x, We, be, Wn, bn, Wexp, bexp, noise_uniform):
    B, C, N, P = x.shape
    O = Wexp.shape[0]
    E = We.shape[0]
    u = noise_uniform.reshape(B, N, P, E).transpose(0, 3, 1, 2)
    out = pl.pallas_call(
        _moe_kernel,
        grid=(B, N // _TN),
        in_specs=[
            pl.BlockSpec((1, C, _TN, P), lambda b, n: (b, 0, n, 0)),
            pl.BlockSpec((O, C), lambda b, n: (0, 0)),
            pl.BlockSpec((O, 1), lambda b, n: (0, 0)),
            pl.BlockSpec((E, C), lambda b, n: (0, 0)),
            pl.BlockSpec((E, C), lambda b, n: (0, 0)),
            pl.BlockSpec((1, E, _TN, P), lambda b, n: (b, 0, n, 0)),
        ],
        out_specs=pl.BlockSpec((1, O, _TN, P), lambda b, n: (b, 0, n, 0)),
        out_shape=jax.ShapeDtypeStruct((B, O, N, P), x.dtype),
        compiler_params=pltpu.CompilerParams(
            dimension_semantics=("parallel", "parallel")),
    )(x, Wexp.astype(jnp.bfloat16), bexp.reshape(O, 1),
      We.astype(jnp.bfloat16), Wn.astype(jnp.bfloat16), u)
    return out


# speedup vs baseline: 1.0433x; 1.0433x over previous
import jax
import jax.numpy as jnp
from jax.experimental import pallas as pl
from jax.experimental.pallas import tpu as pltpu

_TM = 2048


def _moe_kernel(x_ref, w_ref, b_ref, we_ref, wn_ref, u_ref, o_ref):
    xb = x_ref[0]
    y = jax.lax.dot(
        w_ref[...], xb, preferred_element_type=jnp.float32,
    )

    el = jax.lax.dot(we_ref[...], xb, preferred_element_type=jnp.float32)
    nl = jax.lax.dot(wn_ref[...], xb, preferred_element_type=jnp.float32)
    nexp = jnp.exp(nl - jnp.max(nl, axis=0, keepdims=True))
    noise = u_ref[0] * (nexp / jnp.sum(nexp, axis=0, keepdims=True))
    logits = el + noise
    drop = jnp.argmin(logits, axis=0)[None, :]
    keep = jax.lax.broadcasted_iota(jnp.int32, logits.shape, 0) != drop
    mexp = jnp.where(keep, jnp.exp(logits - jnp.max(logits, axis=0, keepdims=True)), 0.0)
    gates = mexp / jnp.sum(mexp, axis=0, keepdims=True)
    gsum = jnp.sum(gates, axis=0, keepdims=True)

    o_ref[0] = gsum * y + b_ref[...]


def kernel(x, We, be, Wn, bn, Wexp, bexp, noise_uniform):
    B, C, N, P = x.shape
    M = N * P
    O = Wexp.shape[0]
    E = We.shape[0]
    xr = x.reshape(B, C, M).astype(jnp.bfloat16)
    u = noise_uniform.reshape(B, M, E).transpose(0, 2, 1)
    out = pl.pallas_call(
        _moe_kernel,
        grid=(B, M // _TM),
        in_specs=[
            pl.BlockSpec((1, C, _TM), lambda b, m: (b, 0, m)),
            pl.BlockSpec((O, C), lambda b, m: (0, 0)),
            pl.BlockSpec((O, 1), lambda b, m: (0, 0)),
            pl.BlockSpec((E, C), lambda b, m: (0, 0)),
            pl.BlockSpec((E, C), lambda b, m: (0, 0)),
            pl.BlockSpec((1, E, _TM), lambda b, m: (b, 0, m)),
        ],
        out_specs=pl.BlockSpec((1, O, _TM), lambda b, m: (b, 0, m)),
        out_shape=jax.ShapeDtypeStruct((B, O, M), x.dtype),
        compiler_params=pltpu.CompilerParams(
            dimension_semantics=("parallel", "parallel")),
    )(xr, Wexp.astype(jnp.bfloat16), bexp.reshape(O, 1),
      We.astype(jnp.bfloat16), Wn.astype(jnp.bfloat16), u)
    return out.reshape(B, O, N, P)
